# baseline (speedup 1.0000x reference)
import functools

import jax
import jax.numpy as jnp
from jax import lax
from jax.experimental import pallas as pl
from jax.experimental.pallas import tpu as pltpu
from jax.experimental.pallas import tpu_sc as plsc

N = 10000
E = 320000
D = 128

NC = 2
NS = 16
C = 100
CD = 125

CH_P = E // (NS * C)
IB = 10
NB = CH_P // IB
CH_D = E // (NC * NS * CD)
ROWS_T = 640
N_PAD = NS * ROWS_T
WD = 128

RB = 1024
NBLK = N_PAD // RB


@functools.cache
def _sc_mesh():
    return plsc.VectorSubcoreMesh(core_axis_name="c", subcore_axis_name="s",
                                  num_cores=NC, num_subcores=NS)


_DGRP = 8


def _deg_body(dst_hbm, ones_hbm, zeros_hbm, out_hbm, dst_v, ones_v, acc, sem):
    c = lax.axis_index("c")
    s = lax.axis_index("s")
    row0 = s * ROWS_T
    pltpu.sync_copy(zeros_hbm.at[s], acc.at[pl.ds(row0, ROWS_T)])
    pltpu.sync_copy(ones_hbm, ones_v)
    pltpu.sync_copy(dst_hbm.at[c].at[s], dst_v)
    plsc.subcore_barrier()

    def group(g, carry):
        j0 = g * _DGRP
        for k in range(_DGRP):
            pltpu.async_copy(ones_v, acc.at[dst_v.at[j0 + k]], sem, add=True)
        for k in range(_DGRP):
            pltpu.make_async_copy(ones_v, acc.at[dst_v.at[j0 + k]], sem).wait()
        return carry

    lax.fori_loop(0, CH_D // _DGRP, group, 0)
    plsc.subcore_barrier()
    pltpu.sync_copy(acc.at[pl.ds(row0, ROWS_T)], out_hbm.at[c].at[s])


@functools.cache
def _deg_kernel_fn():
    return pl.kernel(
        _deg_body,
        out_type=jax.ShapeDtypeStruct((NC, NS, ROWS_T, WD), jnp.float32),
        mesh=_sc_mesh(),
        scratch_types=[
            pltpu.VMEM((CH_D, CD), jnp.int32),
            pltpu.VMEM((CD, WD), jnp.float32),
            pltpu.VMEM_SHARED((N_PAD, WD), jnp.float32),
            pltpu.SemaphoreType.DMA,
        ],
    )


def _deg_kernel(dst_d, ones_c1, zeros_n1):
    out = _deg_kernel_fn()(dst_d, ones_c1, zeros_n1)
    return out.reshape(NC, N_PAD, WD)


def _prop_body(tab, src_hbm, dst_hbm, zeros_hbm, out_hbm,
               src_v, dst_v, rows_0, rows_1, rows_2, acc,
               sem_0, sem_1, sem_2, sem_s, sem_i):
    c = lax.axis_index("c")
    s = lax.axis_index("s")
    row0 = s * ROWS_T
    rows = (rows_0, rows_1, rows_2)
    sems = (sem_0, sem_1, sem_2)
    pltpu.sync_copy(zeros_hbm.at[s], acc.at[pl.ds(row0, ROWS_T)])
    pltpu.sync_copy(src_hbm.at[c].at[s].at[0], src_v.at[0])
    pltpu.sync_copy(dst_hbm.at[s].at[0], dst_v.at[0])
    plsc.subcore_barrier()

    def block(nb, carry):
        sl = lax.rem(nb, 2)
        sln = lax.rem(nb + 1, 2)
        sv = src_v.at[sl]
        dv = dst_v.at[sl]
        pltpu.async_copy(tab.at[sv.at[0]], rows_0, sem_0)
        pltpu.async_copy(tab.at[sv.at[1]], rows_1, sem_1)

        @pl.when(nb + 1 < NB)
        def _():
            pltpu.async_copy(src_hbm.at[c].at[s].at[nb + 1], src_v.at[sln],
                             sem_i)
            pltpu.async_copy(dst_hbm.at[s].at[nb + 1], dst_v.at[sln], sem_i)

        def body(j, carry2):
            r = lax.rem(j, 3)
            for k in range(3):
                @pl.when(r == k)
                def _(k=k):
                    kn = (k + 2) % 3
                    pltpu.make_async_copy(tab.at[sv.at[j]], rows[k],
                                          sems[k]).wait()
                    pltpu.async_copy(rows[k], acc.at[dv.at[j]], sem_s,
                                     add=True)

                    @pl.when(j >= 1)
                    def _():
                        pltpu.make_async_copy(rows[kn],
                                              acc.at[dv.at[j]],
                                              sem_s).wait()

                    @pl.when(j + 2 < IB)
                    def _():
                        pltpu.async_copy(tab.at[sv.at[j + 2]], rows[kn],
                                         sems[kn])
            return carry2

        lax.fori_loop(0, IB, body, carry)
        pltpu.make_async_copy(rows_0, acc.at[dv.at[IB - 1]], sem_s).wait()

        @pl.when(nb + 1 < NB)
        def _():
            pltpu.make_async_copy(src_hbm.at[c].at[s].at[nb + 1],
                                  src_v.at[sln], sem_i).wait()
            pltpu.make_async_copy(dst_hbm.at[s].at[nb + 1],
                                  dst_v.at[sln], sem_i).wait()

        return carry

    lax.fori_loop(0, NB, block, 0)
    plsc.subcore_barrier()
    pltpu.sync_copy(acc.at[pl.ds(row0, ROWS_T)], out_hbm.at[c].at[s])


@functools.cache
def _prop_kernel_fn():
    return pl.kernel(
        _prop_body,
        out_type=jax.ShapeDtypeStruct((NC, NS, ROWS_T, D), jnp.float32),
        mesh=_sc_mesh(),
        scratch_types=[
            pltpu.VMEM((2, IB, C), jnp.int32),
            pltpu.VMEM((2, IB, C), jnp.int32),
            pltpu.VMEM((C, D), jnp.float32),
            pltpu.VMEM((C, D), jnp.float32),
            pltpu.VMEM((C, D), jnp.float32),
            pltpu.VMEM_SHARED((N_PAD, D), jnp.float32),
            pltpu.SemaphoreType.DMA,
            pltpu.SemaphoreType.DMA,
            pltpu.SemaphoreType.DMA,
            pltpu.SemaphoreType.DMA,
            pltpu.SemaphoreType.DMA,
        ],
    )


def _prop_kernel(tab2, src_pc, dst_p, zeros_nd):
    out = _prop_kernel_fn()(tab2, src_pc, dst_p, zeros_nd)
    return out.reshape(NC, N_PAD, D)


def _tc1_body(deg0, deg1, x0, dinv_o, tab_o):
    deg = deg0[0, :, :1] + deg1[0, :, :1] + 1.0
    dinv = lax.rsqrt(deg)
    dinv_o[...] = dinv
    tab_o[0] = x0[...]
    tab_o[1] = x0[...] * dinv


def _tc2_body(x0, acc_a, acc_b, dinv, w1g, b1g, w1c, b1c, tab_o):
    dv = dinv[...]
    hg = jnp.dot(x0[...] + acc_a[0], w1g[...],
                 preferred_element_type=jnp.float32) + b1g[...]
    q = dv * acc_b[0] + (dv * dv) * x0[...]
    h1 = jnp.maximum(jnp.dot(q, w1c[...], preferred_element_type=jnp.float32)
                     + b1c[...], 0.0)
    tab_o[0] = hg
    tab_o[1] = dv * h1


def _tc3_body(hg, tab2b, acc_a, acc_b, dinv,
              w2g, w2c, lin_w, fc_w, b2g, b2c, lin_b, fc_b, out_o):
    wf = jnp.dot(lin_w[...], fc_w[...], preferred_element_type=jnp.float32)
    wa = jnp.dot(w2g[...], wf, preferred_element_type=jnp.float32)
    wb = jnp.dot(w2c[...], wf, preferred_element_type=jnp.float32)
    br = (jnp.dot(b2g[...] + b2c[...], wf, preferred_element_type=jnp.float32)
          + 2.0 * jnp.dot(lin_b[...], fc_w[...],
                          preferred_element_type=jnp.float32)
          + fc_b[...])
    dv = dinv[...]
    xg = hg[0] + acc_a[0]
    p = dv * (acc_b[0] + tab2b[0])
    out_o[...] = (jnp.dot(xg, wa, preferred_element_type=jnp.float32)
                  + jnp.dot(p, wb, preferred_element_type=jnp.float32)
                  + br)


def _full_spec(shape, ndim_grid):
    return pl.BlockSpec(shape, lambda *g: tuple(0 for _ in shape))


def kernel(node_emb, edge_index,
           gin0_W1, gin0_b1, gin0_W2, gin0_b2,
           gin1_W1, gin1_b1, gin1_W2, gin1_b2,
           gcn0_W1, gcn0_b1, gcn0_W2, gcn0_b2,
           gcn1_W1, gcn1_b1, gcn1_W2, gcn1_b2,
           lin_W, lin_b, fc_W, fc_b):
    x0p = jnp.pad(node_emb, ((0, N_PAD - N), (0, 0)))
    src = edge_index[0]
    dst = edge_index[1]

    srcr = src.reshape(NS, NB, IB, C)
    src_pc = jnp.stack([srcr, srcr + N_PAD])
    dst_p = dst.reshape(NS, NB, IB, C)
    dst_d = dst.reshape(NC, NS, CH_D, CD)

    zeros_nd = jnp.zeros((NS, ROWS_T, D), jnp.float32)
    zeros_n1 = jnp.zeros((NS, ROWS_T, WD), jnp.float32)
    ones_c1 = jnp.ones((CD, WD), jnp.float32)

    degs = _deg_kernel(dst_d, ones_c1, zeros_n1)

    row_i = pl.BlockSpec((RB, D), lambda i: (i, 0))
    col_i = pl.BlockSpec((RB, 1), lambda i: (i, 0))
    pair_i = pl.BlockSpec((2, RB, D), lambda i: (0, i, 0))
    dinv, tab1 = pl.pallas_call(
        _tc1_body,
        grid=(NBLK,),
        in_specs=[pl.BlockSpec((1, RB, WD), lambda i: (0, i, 0)),
                  pl.BlockSpec((1, RB, WD), lambda i: (1, i, 0)),
                  row_i],
        out_specs=[col_i, pair_i],
        out_shape=[jax.ShapeDtypeStruct((N_PAD, 1), jnp.float32),
                   jax.ShapeDtypeStruct((2, N_PAD, D), jnp.float32)],
    )(degs, degs, x0p)
    tab1 = tab1.reshape(2 * N_PAD, D)

    acc1 = _prop_kernel(tab1, src_pc, dst_p, zeros_nd)

    tab2 = pl.pallas_call(
        _tc2_body,
        grid=(NBLK,),
        in_specs=[row_i,
                  pl.BlockSpec((1, RB, D), lambda i: (0, i, 0)),
                  pl.BlockSpec((1, RB, D), lambda i: (1, i, 0)),
                  col_i,
                  _full_spec((D, D), 1), _full_spec((1, D), 1),
                  _full_spec((D, D), 1), _full_spec((1, D), 1)],
        out_specs=pair_i,
        out_shape=jax.ShapeDtypeStruct((2, N_PAD, D), jnp.float32),
    )(x0p, acc1, acc1, dinv,
      gin1_W1, gin1_b1.reshape(1, D), gcn1_W1, gcn1_b1.reshape(1, D))

    acc2 = _prop_kernel(tab2.reshape(2 * N_PAD, D), src_pc, dst_p, zeros_nd)

    out = pl.pallas_call(
        _tc3_body,
        grid=(NBLK,),
        in_specs=[pl.BlockSpec((1, RB, D), lambda i: (0, i, 0)),
                  pl.BlockSpec((1, RB, D), lambda i: (1, i, 0)),
                  pl.BlockSpec((1, RB, D), lambda i: (0, i, 0)),
                  pl.BlockSpec((1, RB, D), lambda i: (1, i, 0)),
                  col_i,
                  _full_spec((D, D), 1), _full_spec((D, D), 1),
                  _full_spec((D, D), 1), _full_spec((D, D), 1),
                  _full_spec((1, D), 1), _full_spec((1, D), 1),
                  _full_spec((1, D), 1), _full_spec((1, D), 1)],
        out_specs=row_i,
        out_shape=jax.ShapeDtypeStruct((N_PAD, D), jnp.float32),
    )(tab2, tab2, acc2, acc2, dinv,
      gin1_W2, gcn1_W2, lin_W, fc_W,
      gin1_b2.reshape(1, D), gcn1_b2.reshape(1, D),
      lin_b.reshape(1, D), fc_b.reshape(1, D))

    return out[:N]


# speedup vs baseline: 1.0219x; 1.0219x over previous
import functools

import jax
import jax.numpy as jnp
from jax import lax
from jax.experimental import pallas as pl
from jax.experimental.pallas import tpu as pltpu
from jax.experimental.pallas import tpu_sc as plsc

N = 10000
E = 320000
D = 128

NC = 2
NS = 16
C = 100
CD = 125

CH_P = E // (NS * C)
IB = 25
NB = CH_P // IB
CH_D = E // (NC * NS * CD)
ROWS_T = 640
N_PAD = NS * ROWS_T
WD = 128

RB = 1024
NBLK = N_PAD // RB


@functools.cache
def _sc_mesh():
    return plsc.VectorSubcoreMesh(core_axis_name="c", subcore_axis_name="s",
                                  num_cores=NC, num_subcores=NS)


_DGRP = 8


def _deg_body(dst_hbm, ones_hbm, zeros_hbm, out_hbm, dst_v, ones_v, acc, sem):
    c = lax.axis_index("c")
    s = lax.axis_index("s")
    row0 = s * ROWS_T
    pltpu.sync_copy(zeros_hbm.at[s], acc.at[pl.ds(row0, ROWS_T)])
    pltpu.sync_copy(ones_hbm, ones_v)
    pltpu.sync_copy(dst_hbm.at[c].at[s], dst_v)
    plsc.subcore_barrier()

    def group(g, carry):
        j0 = g * _DGRP
        for k in range(_DGRP):
            pltpu.async_copy(ones_v, acc.at[dst_v.at[j0 + k]], sem, add=True)
        for k in range(_DGRP):
            pltpu.make_async_copy(ones_v, acc.at[dst_v.at[j0 + k]], sem).wait()
        return carry

    lax.fori_loop(0, CH_D // _DGRP, group, 0)
    plsc.subcore_barrier()
    pltpu.sync_copy(acc.at[pl.ds(row0, ROWS_T)], out_hbm.at[c].at[s])


@functools.cache
def _deg_kernel_fn():
    return pl.kernel(
        _deg_body,
        out_type=jax.ShapeDtypeStruct((NC, NS, ROWS_T, WD), jnp.float32),
        mesh=_sc_mesh(),
        scratch_types=[
            pltpu.VMEM((CH_D, CD), jnp.int32),
            pltpu.VMEM((CD, WD), jnp.float32),
            pltpu.VMEM_SHARED((N_PAD, WD), jnp.float32),
            pltpu.SemaphoreType.DMA,
        ],
    )


def _deg_kernel(dst_d, ones_c1, zeros_n1):
    out = _deg_kernel_fn()(dst_d, ones_c1, zeros_n1)
    return out.reshape(NC, N_PAD, WD)


def _prop_body(tab, src_hbm, dst_hbm, zeros_hbm, out_hbm,
               src_v, dst_v, rows_0, rows_1, rows_2, acc,
               sem_0, sem_1, sem_2, sem_s):
    c = lax.axis_index("c")
    s = lax.axis_index("s")
    row0 = s * ROWS_T
    rows = (rows_0, rows_1, rows_2)
    sems = (sem_0, sem_1, sem_2)
    pltpu.sync_copy(zeros_hbm.at[s], acc.at[pl.ds(row0, ROWS_T)])
    plsc.subcore_barrier()

    def block(nb, carry):
        sv = src_v
        dv = dst_v
        pltpu.sync_copy(src_hbm.at[c].at[s].at[nb], src_v)
        pltpu.sync_copy(dst_hbm.at[s].at[nb], dst_v)
        pltpu.async_copy(tab.at[sv.at[0]], rows_0, sem_0)
        pltpu.async_copy(tab.at[sv.at[1]], rows_1, sem_1)

        def body(j, carry2):
            r = lax.rem(j, 3)
            for k in range(3):
                @pl.when(r == k)
                def _(k=k):
                    kn = (k + 2) % 3
                    pltpu.make_async_copy(tab.at[sv.at[j]], rows[k],
                                          sems[k]).wait()
                    pltpu.async_copy(rows[k], acc.at[dv.at[j]], sem_s,
                                     add=True)

                    @pl.when(j >= 1)
                    def _():
                        pltpu.make_async_copy(rows[kn],
                                              acc.at[dv.at[j]],
                                              sem_s).wait()

                    @pl.when(j + 2 < IB)
                    def _():
                        pltpu.async_copy(tab.at[sv.at[j + 2]], rows[kn],
                                         sems[kn])
            return carry2

        lax.fori_loop(0, IB, body, carry)
        pltpu.make_async_copy(rows_0, acc.at[dv.at[IB - 1]], sem_s).wait()
        return carry

    lax.fori_loop(0, NB, block, 0)
    plsc.subcore_barrier()
    pltpu.sync_copy(acc.at[pl.ds(row0, ROWS_T)], out_hbm.at[c].at[s])


@functools.cache
def _prop_kernel_fn():
    return pl.kernel(
        _prop_body,
        out_type=jax.ShapeDtypeStruct((NC, NS, ROWS_T, D), jnp.float32),
        mesh=_sc_mesh(),
        scratch_types=[
            pltpu.VMEM((IB, C), jnp.int32),
            pltpu.VMEM((IB, C), jnp.int32),
            pltpu.VMEM((C, D), jnp.float32),
            pltpu.VMEM((C, D), jnp.float32),
            pltpu.VMEM((C, D), jnp.float32),
            pltpu.VMEM_SHARED((N_PAD, D), jnp.float32),
            pltpu.SemaphoreType.DMA,
            pltpu.SemaphoreType.DMA,
            pltpu.SemaphoreType.DMA,
            pltpu.SemaphoreType.DMA,
        ],
    )


def _prop_kernel(tab2, src_pc, dst_p, zeros_nd):
    out = _prop_kernel_fn()(tab2, src_pc, dst_p, zeros_nd)
    return out.reshape(NC, N_PAD, D)


def _tc1_body(deg0, deg1, x0, dinv_o, tab_o):
    deg = deg0[0, :, :1] + deg1[0, :, :1] + 1.0
    dinv = lax.rsqrt(deg)
    dinv_o[...] = dinv
    tab_o[0] = x0[...]
    tab_o[1] = x0[...] * dinv


def _tc2_body(x0, acc_a, acc_b, dinv, w1g, b1g, w1c, b1c, tab_o):
    dv = dinv[...]
    hg = jnp.dot(x0[...] + acc_a[0], w1g[...],
                 preferred_element_type=jnp.float32) + b1g[...]
    q = dv * acc_b[0] + (dv * dv) * x0[...]
    h1 = jnp.maximum(jnp.dot(q, w1c[...], preferred_element_type=jnp.float32)
                     + b1c[...], 0.0)
    tab_o[0] = hg
    tab_o[1] = dv * h1


def _tc3_body(hg, tab2b, acc_a, acc_b, dinv,
              w2g, w2c, lin_w, fc_w, b2g, b2c, lin_b, fc_b, out_o):
    wf = jnp.dot(lin_w[...], fc_w[...], preferred_element_type=jnp.float32)
    wa = jnp.dot(w2g[...], wf, preferred_element_type=jnp.float32)
    wb = jnp.dot(w2c[...], wf, preferred_element_type=jnp.float32)
    br = (jnp.dot(b2g[...] + b2c[...], wf, preferred_element_type=jnp.float32)
          + 2.0 * jnp.dot(lin_b[...], fc_w[...],
                          preferred_element_type=jnp.float32)
          + fc_b[...])
    dv = dinv[...]
    xg = hg[0] + acc_a[0]
    p = dv * (acc_b[0] + tab2b[0])
    out_o[...] = (jnp.dot(xg, wa, preferred_element_type=jnp.float32)
                  + jnp.dot(p, wb, preferred_element_type=jnp.float32)
                  + br)


def _full_spec(shape, ndim_grid):
    return pl.BlockSpec(shape, lambda *g: tuple(0 for _ in shape))


def kernel(node_emb, edge_index,
           gin0_W1, gin0_b1, gin0_W2, gin0_b2,
           gin1_W1, gin1_b1, gin1_W2, gin1_b2,
           gcn0_W1, gcn0_b1, gcn0_W2, gcn0_b2,
           gcn1_W1, gcn1_b1, gcn1_W2, gcn1_b2,
           lin_W, lin_b, fc_W, fc_b):
    x0p = jnp.pad(node_emb, ((0, N_PAD - N), (0, 0)))
    src = edge_index[0]
    dst = edge_index[1]

    srcr = src.reshape(NS, NB, IB, C)
    src_pc = jnp.stack([srcr, srcr + N_PAD])
    dst_p = dst.reshape(NS, NB, IB, C)
    dst_d = dst.reshape(NC, NS, CH_D, CD)

    zeros_nd = jnp.zeros((NS, ROWS_T, D), jnp.float32)
    zeros_n1 = jnp.zeros((NS, ROWS_T, WD), jnp.float32)
    ones_c1 = jnp.ones((CD, WD), jnp.float32)

    degs = _deg_kernel(dst_d, ones_c1, zeros_n1)

    row_i = pl.BlockSpec((RB, D), lambda i: (i, 0))
    col_i = pl.BlockSpec((RB, 1), lambda i: (i, 0))
    pair_i = pl.BlockSpec((2, RB, D), lambda i: (0, i, 0))
    dinv, tab1 = pl.pallas_call(
        _tc1_body,
        grid=(NBLK,),
        in_specs=[pl.BlockSpec((1, RB, WD), lambda i: (0, i, 0)),
                  pl.BlockSpec((1, RB, WD), lambda i: (1, i, 0)),
                  row_i],
        out_specs=[col_i, pair_i],
        out_shape=[jax.ShapeDtypeStruct((N_PAD, 1), jnp.float32),
                   jax.ShapeDtypeStruct((2, N_PAD, D), jnp.float32)],
    )(degs, degs, x0p)
    tab1 = tab1.reshape(2 * N_PAD, D)

    acc1 = _prop_kernel(tab1, src_pc, dst_p, zeros_nd)

    tab2 = pl.pallas_call(
        _tc2_body,
        grid=(NBLK,),
        in_specs=[row_i,
                  pl.BlockSpec((1, RB, D), lambda i: (0, i, 0)),
                  pl.BlockSpec((1, RB, D), lambda i: (1, i, 0)),
                  col_i,
                  _full_spec((D, D), 1), _full_spec((1, D), 1),
                  _full_spec((D, D), 1), _full_spec((1, D), 1)],
        out_specs=pair_i,
        out_shape=jax.ShapeDtypeStruct((2, N_PAD, D), jnp.float32),
    )(x0p, acc1, acc1, dinv,
      gin1_W1, gin1_b1.reshape(1, D), gcn1_W1, gcn1_b1.reshape(1, D))

    acc2 = _prop_kernel(tab2.reshape(2 * N_PAD, D), src_pc, dst_p, zeros_nd)

    out = pl.pallas_call(
        _tc3_body,
        grid=(NBLK,),
        in_specs=[pl.BlockSpec((1, RB, D), lambda i: (0, i, 0)),
                  pl.BlockSpec((1, RB, D), lambda i: (1, i, 0)),
                  pl.BlockSpec((1, RB, D), lambda i: (0, i, 0)),
                  pl.BlockSpec((1, RB, D), lambda i: (1, i, 0)),
                  col_i,
                  _full_spec((D, D), 1), _full_spec((D, D), 1),
                  _full_spec((D, D), 1), _full_spec((D, D), 1),
                  _full_spec((1, D), 1), _full_spec((1, D), 1),
                  _full_spec((1, D), 1), _full_spec((1, D), 1)],
        out_specs=row_i,
        out_shape=jax.ShapeDtypeStruct((N_PAD, D), jnp.float32),
    )(tab2, tab2, acc2, acc2, dinv,
      gin1_W2, gcn1_W2, lin_W, fc_W,
      gin1_b2.reshape(1, D), gcn1_b2.reshape(1, D),
      lin_b.reshape(1, D), fc_b.reshape(1, D))

    return out[:N]
